# baseline (speedup 1.0000x reference)
import functools

import jax
import jax.numpy as jnp
from jax import lax
from jax.experimental import pallas as pl
from jax.experimental.pallas import tpu as pltpu
from jax.experimental.pallas import tpu_sc as plsc

N_ATOMS = 50000
N_EDGES = 1600000
N_FEATURES = 128
N_BASIS = 16
CUTOFF = 5.0
N_TABLE = 119

_DELTA = CUTOFF / (N_BASIS - 1)
_GAMMA = 1.0 / (2.0 * _DELTA * _DELTA)

_NC, _NS = 2, 16
_NW = _NC * _NS
_B_PER_W = 1568
_LAST_TOTAL = N_ATOMS - (_NW - 1) * _B_PER_W
_CHUNK = 400
_SIZES_FULL = (400, 400, 400, 368)
_SIZES_LAST = (400, 400, 400, 192)
_N_CHUNK = 4


@functools.partial(
    pl.kernel,
    out_type=jax.ShapeDtypeStruct((N_ATOMS * N_FEATURES,), jnp.float32),
    mesh=plsc.VectorSubcoreMesh(core_axis_name="c", subcore_axis_name="s"),
    compiler_params=pltpu.CompilerParams(needs_layout_passes=False),
    scratch_types=[
        pltpu.VMEM((N_TABLE * N_FEATURES,), jnp.float32),
        pltpu.VMEM((_B_PER_W,), jnp.int32),
        pltpu.VMEM((_CHUNK * N_FEATURES,), jnp.float32),
        pltpu.VMEM((_CHUNK * N_FEATURES,), jnp.float32),
        pltpu.SemaphoreType.DMA,
        pltpu.SemaphoreType.DMA,
    ],
)
def _gather_kernel(table_hbm, idx_hbm, out_hbm, table_v, idx_v, buf0, buf1,
                   sem0, sem1):
    wid = lax.axis_index("s") * _NC + lax.axis_index("c")
    base = wid * _B_PER_W
    last = wid == _NW - 1
    not_last = wid < _NW - 1

    pltpu.sync_copy(table_hbm, table_v)

    @pl.when(not_last)
    def _idx_full():
        pltpu.sync_copy(idx_hbm.at[pl.ds(base, _B_PER_W)], idx_v)

    @pl.when(last)
    def _idx_tail():
        pltpu.sync_copy(idx_hbm.at[pl.ds(base, _LAST_TOTAL)],
                        idx_v.at[pl.ds(0, _LAST_TOTAL)])

    bufs = (buf0, buf1)
    sems = (sem0, sem1)
    iota16 = lax.broadcasted_iota(jnp.int32, (16,), 0)

    def _fill(c, size):
        buf = bufs[c % 2]

        def group_body(g, carry):
            i0 = g * 16
            z16 = idx_v[pl.ds(c * _CHUNK + i0, 16)]
            zb = z16 * N_FEATURES
            rowb = (iota16 + i0) * N_FEATURES
            for col in range(N_FEATURES):
                vals = plsc.load_gather(table_v, [zb + col])
                plsc.store_scatter(buf, [rowb + col], vals)
            return carry

        lax.fori_loop(0, size // 16, group_body, 0)

    def _wb_desc(c, size):
        return (bufs[c % 2].at[pl.ds(0, size * N_FEATURES)],
                out_hbm.at[pl.ds((base + c * _CHUNK) * N_FEATURES,
                                 size * N_FEATURES)], sems[c % 2])

    def _branched(c, full, tail):
        if c < _N_CHUNK - 1:
            full()
        else:
            pl.when(not_last)(full)
            pl.when(last)(tail)

    for c in range(_N_CHUNK):
        if c >= 2:
            pltpu.make_async_copy(*_wb_desc(c - 2, _SIZES_FULL[c - 2])).wait()

        def _full(cc=c):
            _fill(cc, _SIZES_FULL[cc])
            pltpu.async_copy(*_wb_desc(cc, _SIZES_FULL[cc]))

        def _tail(cc=c):
            _fill(cc, _SIZES_LAST[cc])
            pltpu.async_copy(*_wb_desc(cc, _SIZES_LAST[cc]))

        _branched(c, _full, _tail)

    def _drain_full():
        pltpu.make_async_copy(*_wb_desc(2, _SIZES_FULL[2])).wait()
        pltpu.make_async_copy(*_wb_desc(3, _SIZES_FULL[3])).wait()

    def _drain_tail():
        pltpu.make_async_copy(*_wb_desc(2, _SIZES_LAST[2])).wait()
        pltpu.make_async_copy(*_wb_desc(3, _SIZES_LAST[3])).wait()

    pl.when(not_last)(_drain_full)
    pl.when(last)(_drain_tail)


_BT = 64000
_GRID = N_EDGES // _BT
_BZ = (-(-N_ATOMS // _GRID) + 7) // 8 * 8


def _edge_body(dispT_ref, dirT_ref, distT_ref, zf_ref, zd_ref):
    x = dispT_ref[0:1, :]
    y = dispT_ref[1:2, :]
    z = dispT_ref[2:3, :]
    n2 = x * x + y * y + z * z + 1e-12
    inv = lax.rsqrt(n2)
    dist = n2 * inv
    dirT_ref[0:1, :] = x * inv
    dirT_ref[1:2, :] = y * inv
    dirT_ref[2:3, :] = z * inv
    cut = 0.5 * (jnp.cos((jnp.pi / CUTOFF) * dist) + 1.0)
    cut = jnp.where(dist < CUTOFF, cut, 0.0)
    centers = lax.broadcasted_iota(jnp.int32, (N_BASIS, 1), 0).astype(
        jnp.float32) * _DELTA
    diff = dist - centers
    distT_ref[...] = cut * jnp.exp(-_GAMMA * (diff * diff))
    zf_ref[...] = jnp.zeros((3, _BZ, N_FEATURES), jnp.float32)
    zd_ref[...] = jnp.zeros((3, _BZ, N_FEATURES), jnp.float32)


_edge_call = pl.pallas_call(
    _edge_body,
    grid=(_GRID,),
    in_specs=[pl.BlockSpec((3, _BT), lambda i: (0, i))],
    out_specs=[
        pl.BlockSpec((3, _BT), lambda i: (0, i)),
        pl.BlockSpec((N_BASIS, _BT), lambda i: (0, i)),
        pl.BlockSpec((3, _BZ, N_FEATURES), lambda i: (0, i, 0)),
        pl.BlockSpec((3, _BZ, N_FEATURES), lambda i: (0, i, 0)),
    ],
    out_shape=[
        jax.ShapeDtypeStruct((3, N_EDGES), jnp.float32),
        jax.ShapeDtypeStruct((N_BASIS, N_EDGES), jnp.float32),
        jax.ShapeDtypeStruct((3, N_ATOMS, N_FEATURES), jnp.float32),
        jax.ShapeDtypeStruct((3, N_ATOMS, N_FEATURES), jnp.float32),
    ],
)


def kernel(z, disp, node_table):
    atom_node = _gather_kernel(node_table.reshape(-1),
                               z.astype(jnp.int32)).reshape(N_ATOMS,
                                                            N_FEATURES)
    dirT, distT, zf, zd = _edge_call(disp.T)
    dir_edge = dirT.T
    dist_edge = distT.T
    force_node = jnp.transpose(zf, (1, 0, 2))
    disp_node = jnp.transpose(zd, (1, 0, 2))
    return (atom_node, force_node, disp_node, dir_edge, dist_edge)


# speedup vs baseline: 2.2694x; 2.2694x over previous
import functools

import jax
import jax.numpy as jnp
from jax import lax
from jax.experimental import pallas as pl
from jax.experimental.pallas import tpu as pltpu
from jax.experimental.pallas import tpu_sc as plsc

N_ATOMS = 50000
N_EDGES = 1600000
N_FEATURES = 128
N_BASIS = 16
CUTOFF = 5.0

_DELTA = CUTOFF / (N_BASIS - 1)
_GAMMA = 1.0 / (2.0 * _DELTA * _DELTA)

_NC, _NS = 2, 16
_NW = _NC * _NS
_B_PER_W = 1568
_N_CHUNK = 4
_CHUNK = _B_PER_W // _N_CHUNK
_LAST_TOTAL = N_ATOMS - (_NW - 1) * _B_PER_W
_TAIL = _LAST_TOTAL - (_N_CHUNK - 1) * _CHUNK


@functools.partial(
    pl.kernel,
    out_type=jax.ShapeDtypeStruct((N_ATOMS, N_FEATURES), jnp.float32),
    mesh=plsc.VectorSubcoreMesh(core_axis_name="c", subcore_axis_name="s"),
    scratch_types=[
        pltpu.VMEM_SHARED((119, N_FEATURES), jnp.float32),
        pltpu.VMEM((_B_PER_W,), jnp.int32),
        pltpu.VMEM((_CHUNK, N_FEATURES), jnp.float32),
        pltpu.VMEM((_CHUNK, N_FEATURES), jnp.float32),
        pltpu.SemaphoreType.DMA,
        pltpu.SemaphoreType.DMA,
    ],
)
def _gather_kernel(table_hbm, idx_hbm, out_hbm, table_v, idx_v, rows0, rows1,
                   sem0, sem1):
    wid = lax.axis_index("s") * _NC + lax.axis_index("c")

    @pl.when(lax.axis_index("s") == 0)
    def _load_table():
        pltpu.sync_copy(table_hbm, table_v)

    plsc.subcore_barrier()
    base = wid * _B_PER_W
    last = wid == _NW - 1
    not_last = wid < _NW - 1

    @pl.when(not_last)
    def _idx_full():
        pltpu.sync_copy(idx_hbm.at[pl.ds(base, _B_PER_W)], idx_v)

    @pl.when(last)
    def _idx_tail():
        pltpu.sync_copy(idx_hbm.at[pl.ds(base, _LAST_TOTAL)],
                        idx_v.at[pl.ds(0, _LAST_TOTAL)])

    bufs = (rows0, rows1)
    sems = (sem0, sem1)

    def _branched(c, full, tail):
        if c < _N_CHUNK - 1:
            full()
        else:
            pl.when(not_last)(full)
            pl.when(last)(tail)

    def _gather_desc(c, size):
        b = c % 2
        return (table_v.at[idx_v.at[pl.ds(c * _CHUNK, size)]],
                bufs[b].at[pl.ds(0, size)], sems[b])

    def start(c):
        def _full():
            pltpu.async_copy(*_gather_desc(c, _CHUNK))

        def _tail():
            pltpu.async_copy(*_gather_desc(c, _TAIL))

        _branched(c, _full, _tail)

    def finish(c):
        b = c % 2

        def _fin(size):
            pltpu.make_async_copy(*_gather_desc(c, size)).wait()
            pltpu.sync_copy(bufs[b].at[pl.ds(0, size)],
                            out_hbm.at[pl.ds(base + c * _CHUNK, size)])

        def _full():
            _fin(_CHUNK)

        def _tail():
            _fin(_TAIL)

        _branched(c, _full, _tail)

    start(0)
    for c in range(_N_CHUNK):
        if c + 1 < _N_CHUNK:
            start(c + 1)
        finish(c)


_BT = 64000
_GRID = N_EDGES // _BT
_BZ = (-(-N_ATOMS // _GRID) + 7) // 8 * 8


def _edge_body(dispT_ref, dirT_ref, distT_ref, zf_ref, zd_ref):
    x = dispT_ref[0:1, :]
    y = dispT_ref[1:2, :]
    z = dispT_ref[2:3, :]
    n2 = x * x + y * y + z * z + 1e-12
    inv = lax.rsqrt(n2)
    dist = n2 * inv
    dirT_ref[0:1, :] = x * inv
    dirT_ref[1:2, :] = y * inv
    dirT_ref[2:3, :] = z * inv
    cut = 0.5 * (jnp.cos((jnp.pi / CUTOFF) * dist) + 1.0)
    cut = jnp.where(dist < CUTOFF, cut, 0.0)
    centers = lax.broadcasted_iota(jnp.int32, (N_BASIS, 1), 0).astype(
        jnp.float32) * _DELTA
    diff = dist - centers
    distT_ref[...] = cut * jnp.exp(-_GAMMA * (diff * diff))
    zf_ref[...] = jnp.zeros((3, _BZ, N_FEATURES), jnp.float32)
    zd_ref[...] = jnp.zeros((3, _BZ, N_FEATURES), jnp.float32)


_edge_call = pl.pallas_call(
    _edge_body,
    grid=(N_EDGES // _BT,),
    in_specs=[pl.BlockSpec((3, _BT), lambda i: (0, i))],
    out_specs=[
        pl.BlockSpec((3, _BT), lambda i: (0, i)),
        pl.BlockSpec((N_BASIS, _BT), lambda i: (0, i)),
        pl.BlockSpec((3, _BZ, N_FEATURES), lambda i: (0, i, 0)),
        pl.BlockSpec((3, _BZ, N_FEATURES), lambda i: (0, i, 0)),
    ],
    out_shape=[
        jax.ShapeDtypeStruct((3, N_EDGES), jnp.float32),
        jax.ShapeDtypeStruct((N_BASIS, N_EDGES), jnp.float32),
        jax.ShapeDtypeStruct((3, N_ATOMS, N_FEATURES), jnp.float32),
        jax.ShapeDtypeStruct((3, N_ATOMS, N_FEATURES), jnp.float32),
    ],
)


def kernel(z, disp, node_table):
    atom_node = _gather_kernel(node_table, z.astype(jnp.int32))
    dirT, distT, zf, zd = _edge_call(disp.T)
    dir_edge = dirT.T
    dist_edge = distT.T
    force_node = jnp.transpose(zf, (1, 0, 2))
    disp_node = jnp.transpose(zd, (1, 0, 2))
    return (atom_node, force_node, disp_node, dir_edge, dist_edge)
